# baseline (speedup 1.0000x reference)
import jax
import jax.numpy as jnp
from jax.experimental import pallas as pl
from jax.experimental.pallas import tpu as pltpu

_BLOCK = 256
_TOTAL_BL = 4096


def _bcast_dma_kernel(emb_ref, out_ref, scratch_ref, sems_ref):
    n_copies = _TOTAL_BL // _BLOCK
    scratch_ref[...] = jnp.broadcast_to(emb_ref[...][None], scratch_ref.shape)
    for i in range(n_copies):
        pltpu.make_async_copy(
            scratch_ref,
            out_ref.at[pl.ds(i * _BLOCK, _BLOCK)],
            sems_ref.at[i],
        ).start()
    for i in range(n_copies):
        pltpu.make_async_copy(
            scratch_ref,
            out_ref.at[pl.ds(i * _BLOCK, _BLOCK)],
            sems_ref.at[i],
        ).wait()


def kernel(x, var_emb):
    B, L, D = x.shape
    d_model = var_emb.shape[1]
    BL = B * L
    emb = var_emb[:D]

    out = pl.pallas_call(
        _bcast_dma_kernel,
        in_specs=[pl.BlockSpec(memory_space=pltpu.VMEM)],
        out_specs=pl.BlockSpec(memory_space=pl.ANY),
        out_shape=jax.ShapeDtypeStruct((BL, D, d_model), var_emb.dtype),
        scratch_shapes=[
            pltpu.VMEM((_BLOCK, D, d_model), var_emb.dtype),
            pltpu.SemaphoreType.DMA((BL // _BLOCK,)),
        ],
    )(emb)
    return out.reshape(B, L, D, d_model)
